# Initial kernel scaffold; baseline (speedup 1.0000x reference)
import functools

import jax
import jax.numpy as jnp
from jax import lax
from jax.experimental import pallas as pl
from jax.experimental.pallas import tpu as pltpu
from jax.experimental.pallas import tpu_sc as plsc

f32 = jnp.float32
i32 = jnp.int32

NC = 2
NS = 16
NW = NC * NS
L = 16


def _make_deg_kernel(E, NP, CH):
    EPT = E // NW
    NIT = EPT // CH
    RPT = NP // NS
    mesh = plsc.VectorSubcoreMesh(core_axis_name="c", subcore_axis_name="s")

    @functools.partial(
        pl.kernel,
        out_type=jax.ShapeDtypeStruct((NC, NP), f32),
        mesh=mesh,
        scratch_types=[
            pltpu.VMEM((CH,), i32),
            pltpu.VMEM((CH,), f32),
            pltpu.VMEM_SHARED((NP,), f32),
        ],
    )
    def deg_kernel(dst_hbm, w_hbm, zeros_hbm, out_hbm, dst_v, w_v, sh_deg):
        c = lax.axis_index("c")
        s = lax.axis_index("s")
        wid = c * NS + s
        pltpu.sync_copy(zeros_hbm.at[pl.ds(s * RPT, RPT)],
                        sh_deg.at[pl.ds(s * RPT, RPT)])
        plsc.subcore_barrier()
        base = wid * EPT

        def body(i, carry):
            off = base + i * CH
            pltpu.sync_copy(dst_hbm.at[pl.ds(off, CH)], dst_v)
            pltpu.sync_copy(w_hbm.at[pl.ds(off, CH)], w_v)
            pltpu.sync_copy(w_v, sh_deg.at[dst_v], add=True)
            return carry

        lax.fori_loop(0, NIT, body, 0)
        plsc.subcore_barrier()
        pltpu.sync_copy(sh_deg.at[pl.ds(s * RPT, RPT)],
                        out_hbm.at[c, pl.ds(s * RPT, RPT)])

    return deg_kernel


def _make_agg_kernel(E, NP, D, CH):
    EPT = E // NW
    NIT = EPT // CH
    RPT = NP // NS
    mesh = plsc.VectorSubcoreMesh(core_axis_name="c", subcore_axis_name="s")

    @functools.partial(
        pl.kernel,
        out_type=jax.ShapeDtypeStruct((NC, NP, D), f32),
        mesh=mesh,
        scratch_types=[
            pltpu.VMEM((CH,), i32),
            pltpu.VMEM((CH,), i32),
            pltpu.VMEM((CH,), f32),
            pltpu.VMEM((CH, D), f32),
            pltpu.VMEM_SHARED((NP, D), f32),
            pltpu.SemaphoreType.DMA,
        ],
    )
    def agg_kernel(y_hbm, src_hbm, dst_hbm, w_hbm, zeros_hbm, out_hbm,
                   src_v, dst_v, w_v, rows_v, sh_acc, sem):
        c = lax.axis_index("c")
        s = lax.axis_index("s")
        wid = c * NS + s
        pltpu.sync_copy(zeros_hbm, sh_acc.at[pl.ds(s * RPT, RPT)])
        plsc.subcore_barrier()
        base = wid * EPT

        def body(i, carry):
            off = base + i * CH
            pltpu.sync_copy(src_hbm.at[pl.ds(off, CH)], src_v)
            pltpu.sync_copy(dst_hbm.at[pl.ds(off, CH)], dst_v)
            pltpu.sync_copy(w_hbm.at[pl.ds(off, CH)], w_v)
            pltpu.async_copy(y_hbm.at[src_v], rows_v, sem).wait()
            for j in range(CH):
                wj = plsc.load_gather(w_v, [jnp.full((L,), j, dtype=i32)])
                for cg in range(D // L):
                    sl = pl.ds(cg * L, L)
                    rows_v[j, sl] = rows_v[j, sl] * wj
            pltpu.sync_copy(rows_v, sh_acc.at[dst_v], add=True)
            return carry

        lax.fori_loop(0, NIT, body, 0)
        plsc.subcore_barrier()
        pltpu.sync_copy(sh_acc.at[pl.ds(s * RPT, RPT)],
                        out_hbm.at[c, pl.ds(s * RPT, RPT)])

    return agg_kernel


def _tc_first_body(degp_ref, x_ref, w_ref, y_ref):
    dinv = lax.rsqrt(degp_ref[0, :] + degp_ref[1, :] + 2.0)
    xw = jnp.dot(x_ref[...], w_ref[...], preferred_element_type=f32)
    y_ref[...] = xw * dinv[:, None]


def _tc_mid_body(degp_ref, a0_ref, a1_ref, y_ref, b_ref, w_ref, y1_ref):
    dinv = lax.rsqrt(degp_ref[0, :] + degp_ref[1, :] + 2.0)[:, None]
    pre = dinv * (a0_ref[...] + a1_ref[...] + 2.0 * y_ref[...]) + b_ref[...]
    h = jnp.maximum(pre, 0.0)
    xw = jnp.dot(h, w_ref[...], preferred_element_type=f32)
    y1_ref[...] = xw * dinv


def _tc_last_body(degp_ref, a0_ref, a1_ref, y_ref, b_ref, out_ref):
    dinv = lax.rsqrt(degp_ref[0, :] + degp_ref[1, :] + 2.0)[:, None]
    pre = dinv * (a0_ref[...] + a1_ref[...] + 2.0 * y_ref[...]) + b_ref[...]
    out_ref[...] = jnp.maximum(pre, 0.0)


def _row_specs(D, BR):
    deg_spec = pl.BlockSpec((NC, BR), lambda i: (0, i))
    mat_spec = pl.BlockSpec((BR, D), lambda i: (i, 0))
    w_spec = pl.BlockSpec((D, D), lambda i: (0, 0))
    b_spec = pl.BlockSpec((1, D), lambda i: (0, 0))
    return deg_spec, mat_spec, w_spec, b_spec


def _tc_first(degp, x_p, W, NP, D, BR):
    deg_spec, mat_spec, w_spec, _ = _row_specs(D, BR)
    return pl.pallas_call(
        _tc_first_body,
        grid=(NP // BR,),
        in_specs=[deg_spec, mat_spec, w_spec],
        out_specs=mat_spec,
        out_shape=jax.ShapeDtypeStruct((NP, D), f32),
    )(degp, x_p, W)


def _tc_mid(degp, a0, a1, y, b2d, W, NP, D, BR):
    deg_spec, mat_spec, w_spec, b_spec = _row_specs(D, BR)
    return pl.pallas_call(
        _tc_mid_body,
        grid=(NP // BR,),
        in_specs=[deg_spec, mat_spec, mat_spec, mat_spec, b_spec, w_spec],
        out_specs=mat_spec,
        out_shape=jax.ShapeDtypeStruct((NP, D), f32),
    )(degp, a0, a1, y, b2d, W)


def _tc_last(degp, a0, a1, y, b2d, NP, D, BR):
    deg_spec, mat_spec, _, b_spec = _row_specs(D, BR)
    return pl.pallas_call(
        _tc_last_body,
        grid=(NP // BR,),
        in_specs=[deg_spec, mat_spec, mat_spec, mat_spec, b_spec],
        out_specs=mat_spec,
        out_shape=jax.ShapeDtypeStruct((NP, D), f32),
    )(degp, a0, a1, y, b2d)


@jax.jit
def kernel(x, edge_index, edge_attr, W0, b0, W1, b1):
    N, D = x.shape
    E = edge_attr.shape[0]
    CH = 80
    NP = -(-N // (NS * 40)) * (NS * 40)
    BR = 512

    src = edge_index[0]
    dst = edge_index[1]
    x_p = jnp.pad(x, ((0, NP - N), (0, 0)))
    zeros1 = jnp.zeros((NP,), f32)
    zeros2 = jnp.zeros((NP // NS, D), f32)

    deg_kernel = _make_deg_kernel(E, NP, CH)
    agg_kernel = _make_agg_kernel(E, NP, D, CH)

    degp = deg_kernel(dst, edge_attr, zeros1)
    y0 = _tc_first(degp, x_p, W0, NP, D, BR)
    accp0 = agg_kernel(y0, src, dst, edge_attr, zeros2)
    y1 = _tc_mid(degp, accp0[0], accp0[1], y0,
                 b0.reshape(1, D), W1, NP, D, BR)
    accp1 = agg_kernel(y1, src, dst, edge_attr, zeros2)
    out = _tc_last(degp, accp1[0], accp1[1], y1,
                   b1.reshape(1, D), NP, D, BR)
    return out[:N]


# speedup vs baseline: 9.6009x; 9.6009x over previous
import functools

import jax
import jax.numpy as jnp
from jax import lax
from jax.experimental import pallas as pl
from jax.experimental.pallas import tpu as pltpu
from jax.experimental.pallas import tpu_sc as plsc

f32 = jnp.float32
i32 = jnp.int32

NC = 2
NS = 16
NW = NC * NS
L = 16


def _make_deg_kernel(E, NP, CH):
    EPT = E // NW
    NIT = EPT // CH
    RPT = NP // NS
    mesh = plsc.VectorSubcoreMesh(core_axis_name="c", subcore_axis_name="s")

    @functools.partial(
        pl.kernel,
        out_type=jax.ShapeDtypeStruct((NC, NP), f32),
        mesh=mesh,
        scratch_types=[
            pltpu.VMEM((CH,), i32),
            pltpu.VMEM((CH,), f32),
            pltpu.VMEM_SHARED((NP,), f32),
        ],
    )
    def deg_kernel(dst_hbm, w_hbm, zeros_hbm, out_hbm, dst_v, w_v, sh_deg):
        c = lax.axis_index("c")
        s = lax.axis_index("s")
        wid = c * NS + s
        pltpu.sync_copy(zeros_hbm.at[pl.ds(s * RPT, RPT)],
                        sh_deg.at[pl.ds(s * RPT, RPT)])
        plsc.subcore_barrier()
        base = wid * EPT

        def body(i, carry):
            off = base + i * CH
            pltpu.sync_copy(dst_hbm.at[pl.ds(off, CH)], dst_v)
            pltpu.sync_copy(w_hbm.at[pl.ds(off, CH)], w_v)
            pltpu.sync_copy(w_v, sh_deg.at[dst_v], add=True)
            return carry

        lax.fori_loop(0, NIT, body, 0)
        plsc.subcore_barrier()
        pltpu.sync_copy(sh_deg.at[pl.ds(s * RPT, RPT)],
                        out_hbm.at[c, pl.ds(s * RPT, RPT)])

    return deg_kernel


def _make_agg_kernel(E, NP, D, CH):
    EPT = E // NW
    NIT = EPT // CH
    RPT = NP // NS
    mesh = plsc.VectorSubcoreMesh(core_axis_name="c", subcore_axis_name="s")

    @functools.partial(
        pl.kernel,
        out_type=jax.ShapeDtypeStruct((NC, NP, D), f32),
        mesh=mesh,
        scratch_types=[
            pltpu.VMEM((CH,), i32),
            pltpu.VMEM((CH,), i32),
            pltpu.VMEM((CH,), f32),
            pltpu.VMEM((CH, D), f32),
            pltpu.VMEM_SHARED((NP, D), f32),
            pltpu.SemaphoreType.DMA,
        ],
    )
    def agg_kernel(y_hbm, src_hbm, dst_hbm, w_hbm, zeros_hbm, out_hbm,
                   src_v, dst_v, w_v, rows_v, sh_acc, sem):
        c = lax.axis_index("c")
        s = lax.axis_index("s")
        wid = c * NS + s
        pltpu.sync_copy(zeros_hbm, sh_acc.at[pl.ds(s * RPT, RPT)])
        plsc.subcore_barrier()
        base = wid * EPT

        def body(i, carry):
            off = base + i * CH
            pltpu.sync_copy(src_hbm.at[pl.ds(off, CH)], src_v)
            pltpu.sync_copy(dst_hbm.at[pl.ds(off, CH)], dst_v)
            pltpu.sync_copy(w_hbm.at[pl.ds(off, CH)], w_v)
            pltpu.async_copy(y_hbm.at[src_v], rows_v, sem).wait()
            for g in range(CH // L):
                w16 = w_v[pl.ds(g * L, L)]
                for jl in range(L):
                    j = g * L + jl
                    wj = w16.at[jnp.full((L,), jl, dtype=i32)].get(
                        mode="promise_in_bounds")
                    for cg in range(D // L):
                        sl = pl.ds(cg * L, L)
                        rows_v[j, sl] = rows_v[j, sl] * wj
            pltpu.sync_copy(rows_v, sh_acc.at[dst_v], add=True)
            return carry

        lax.fori_loop(0, NIT, body, 0)
        plsc.subcore_barrier()
        pltpu.sync_copy(sh_acc.at[pl.ds(s * RPT, RPT)],
                        out_hbm.at[c, pl.ds(s * RPT, RPT)])

    return agg_kernel


def _tc_first_body(degp_ref, x_ref, w_ref, y_ref):
    dinv = lax.rsqrt(degp_ref[0, :] + degp_ref[1, :] + 2.0)
    xw = jnp.dot(x_ref[...], w_ref[...], preferred_element_type=f32)
    y_ref[...] = xw * dinv[:, None]


def _tc_mid_body(degp_ref, a0_ref, a1_ref, y_ref, b_ref, w_ref, y1_ref):
    dinv = lax.rsqrt(degp_ref[0, :] + degp_ref[1, :] + 2.0)[:, None]
    pre = dinv * (a0_ref[...] + a1_ref[...] + 2.0 * y_ref[...]) + b_ref[...]
    h = jnp.maximum(pre, 0.0)
    xw = jnp.dot(h, w_ref[...], preferred_element_type=f32)
    y1_ref[...] = xw * dinv


def _tc_last_body(degp_ref, a0_ref, a1_ref, y_ref, b_ref, out_ref):
    dinv = lax.rsqrt(degp_ref[0, :] + degp_ref[1, :] + 2.0)[:, None]
    pre = dinv * (a0_ref[...] + a1_ref[...] + 2.0 * y_ref[...]) + b_ref[...]
    out_ref[...] = jnp.maximum(pre, 0.0)


def _row_specs(D, BR):
    deg_spec = pl.BlockSpec((NC, BR), lambda i: (0, i))
    mat_spec = pl.BlockSpec((BR, D), lambda i: (i, 0))
    w_spec = pl.BlockSpec((D, D), lambda i: (0, 0))
    b_spec = pl.BlockSpec((1, D), lambda i: (0, 0))
    return deg_spec, mat_spec, w_spec, b_spec


def _tc_first(degp, x_p, W, NP, D, BR):
    deg_spec, mat_spec, w_spec, _ = _row_specs(D, BR)
    return pl.pallas_call(
        _tc_first_body,
        grid=(NP // BR,),
        in_specs=[deg_spec, mat_spec, w_spec],
        out_specs=mat_spec,
        out_shape=jax.ShapeDtypeStruct((NP, D), f32),
    )(degp, x_p, W)


def _tc_mid(degp, a0, a1, y, b2d, W, NP, D, BR):
    deg_spec, mat_spec, w_spec, b_spec = _row_specs(D, BR)
    return pl.pallas_call(
        _tc_mid_body,
        grid=(NP // BR,),
        in_specs=[deg_spec, mat_spec, mat_spec, mat_spec, b_spec, w_spec],
        out_specs=mat_spec,
        out_shape=jax.ShapeDtypeStruct((NP, D), f32),
    )(degp, a0, a1, y, b2d, W)


def _tc_last(degp, a0, a1, y, b2d, NP, D, BR):
    deg_spec, mat_spec, _, b_spec = _row_specs(D, BR)
    return pl.pallas_call(
        _tc_last_body,
        grid=(NP // BR,),
        in_specs=[deg_spec, mat_spec, mat_spec, mat_spec, b_spec],
        out_specs=mat_spec,
        out_shape=jax.ShapeDtypeStruct((NP, D), f32),
    )(degp, a0, a1, y, b2d)


@jax.jit
def kernel(x, edge_index, edge_attr, W0, b0, W1, b1):
    N, D = x.shape
    E = edge_attr.shape[0]
    CH = 80
    NP = -(-N // (NS * 40)) * (NS * 40)
    BR = 512

    src = edge_index[0]
    dst = edge_index[1]
    x_p = jnp.pad(x, ((0, NP - N), (0, 0)))
    zeros1 = jnp.zeros((NP,), f32)
    zeros2 = jnp.zeros((NP // NS, D), f32)

    deg_kernel = _make_deg_kernel(E, NP, CH)
    agg_kernel = _make_agg_kernel(E, NP, D, CH)

    degp = deg_kernel(dst, edge_attr, zeros1)
    y0 = _tc_first(degp, x_p, W0, NP, D, BR)
    accp0 = agg_kernel(y0, src, dst, edge_attr, zeros2)
    y1 = _tc_mid(degp, accp0[0], accp0[1], y0,
                 b0.reshape(1, D), W1, NP, D, BR)
    accp1 = agg_kernel(y1, src, dst, edge_attr, zeros2)
    out = _tc_last(degp, accp1[0], accp1[1], y1,
                   b1.reshape(1, D), NP, D, BR)
    return out[:N]


# speedup vs baseline: 17.8219x; 1.8563x over previous
import functools

import jax
import jax.numpy as jnp
from jax import lax
from jax.experimental import pallas as pl
from jax.experimental.pallas import tpu as pltpu
from jax.experimental.pallas import tpu_sc as plsc

f32 = jnp.float32
i32 = jnp.int32

NC = 2
NS = 16
NW = NC * NS
L = 16
NB = 4


def _make_deg_kernel(E, NP, CH):
    EPT = E // NW
    NIT = EPT // CH
    RPT = NP // NS
    mesh = plsc.VectorSubcoreMesh(core_axis_name="c", subcore_axis_name="s")

    @functools.partial(
        pl.kernel,
        out_type=jax.ShapeDtypeStruct((NC, NP), f32),
        mesh=mesh,
        scratch_types=[
            [pltpu.VMEM((CH,), i32)] * NB,
            [pltpu.VMEM((CH,), f32)] * NB,
            pltpu.VMEM_SHARED((NP,), f32),
            [pltpu.SemaphoreType.DMA] * NB,
            [pltpu.SemaphoreType.DMA] * NB,
        ],
    )
    def deg_kernel(dst_hbm, w_hbm, zeros_hbm, out_hbm, dst_v, w_v, sh_deg,
                   isem, ssem):
        c = lax.axis_index("c")
        s = lax.axis_index("s")
        wid = c * NS + s
        pltpu.sync_copy(zeros_hbm.at[pl.ds(s * RPT, RPT)],
                        sh_deg.at[pl.ds(s * RPT, RPT)])
        plsc.subcore_barrier()
        base = wid * EPT

        def idx_start(ci, b):
            off = base + ci * CH
            pltpu.async_copy(dst_hbm.at[pl.ds(off, CH)], dst_v[b], isem[b])
            pltpu.async_copy(w_hbm.at[pl.ds(off, CH)], w_v[b], isem[b])

        def idx_wait(b):
            pltpu.make_async_copy(dst_hbm.at[pl.ds(0, CH)], dst_v[b],
                                  isem[b]).wait()
            pltpu.make_async_copy(w_hbm.at[pl.ds(0, CH)], w_v[b],
                                  isem[b]).wait()

        def fire(b):
            pltpu.async_copy(w_v[b], sh_deg.at[dst_v[b]], ssem[b], add=True)

        def drain(b):
            pltpu.make_async_copy(w_hbm.at[pl.ds(0, CH)], w_v[b],
                                  ssem[b]).wait()

        for b in range(min(2, NIT)):
            idx_start(b, b)

        def step(ci, b):
            nxt = ci + 2
            bn = (b + 2) % NB

            @pl.when(nxt < NIT)
            def _():
                @pl.when(ci >= 2)
                def _():
                    drain(bn)

                idx_start(nxt, bn)

            idx_wait(b)
            fire(b)

        def body(k, carry):
            for b in range(NB):
                step(k * NB + b, b)
            return carry

        lax.fori_loop(0, NIT // NB, body, 0)
        for t in range(NIT % NB):
            ci = (NIT // NB) * NB + t
            step(ci, ci % NB)
        for b in range(NB):
            drain(b)
        plsc.subcore_barrier()
        pltpu.sync_copy(sh_deg.at[pl.ds(s * RPT, RPT)],
                        out_hbm.at[c, pl.ds(s * RPT, RPT)])

    return deg_kernel


def _make_agg_kernel(E, NP, D, CH):
    EPT = E // NW
    NIT = EPT // CH
    RPT = NP // NS
    mesh = plsc.VectorSubcoreMesh(core_axis_name="c", subcore_axis_name="s")

    @functools.partial(
        pl.kernel,
        out_type=jax.ShapeDtypeStruct((NC, NP, D), f32),
        mesh=mesh,
        scratch_types=[
            [pltpu.VMEM((CH,), i32)] * NB,
            [pltpu.VMEM((CH,), i32)] * NB,
            [pltpu.VMEM((CH,), f32)] * NB,
            [pltpu.VMEM((CH, D), f32)] * NB,
            pltpu.VMEM_SHARED((NP, D), f32),
            [pltpu.SemaphoreType.DMA] * NB,
            [pltpu.SemaphoreType.DMA] * NB,
            [pltpu.SemaphoreType.DMA] * NB,
        ],
    )
    def agg_kernel(y_hbm, src_hbm, dst_hbm, w_hbm, zeros_hbm, out_hbm,
                   src_v, dst_v, w_v, rows, sh_acc, isem, gsem, ssem):
        c = lax.axis_index("c")
        s = lax.axis_index("s")
        wid = c * NS + s
        pltpu.sync_copy(zeros_hbm, sh_acc.at[pl.ds(s * RPT, RPT)])
        plsc.subcore_barrier()
        base = wid * EPT

        def idx_start(ci, b):
            off = base + ci * CH
            pltpu.async_copy(src_hbm.at[pl.ds(off, CH)], src_v[b], isem[b])
            pltpu.async_copy(dst_hbm.at[pl.ds(off, CH)], dst_v[b], isem[b])
            pltpu.async_copy(w_hbm.at[pl.ds(off, CH)], w_v[b], isem[b])

        def idx_wait(b):
            pltpu.make_async_copy(src_hbm.at[pl.ds(0, CH)], src_v[b],
                                  isem[b]).wait()
            pltpu.make_async_copy(dst_hbm.at[pl.ds(0, CH)], dst_v[b],
                                  isem[b]).wait()
            pltpu.make_async_copy(w_hbm.at[pl.ds(0, CH)], w_v[b],
                                  isem[b]).wait()

        def gather_start(b):
            pltpu.async_copy(y_hbm.at[src_v[b]], rows[b], gsem[b])

        def gather_wait(b):
            pltpu.make_async_copy(y_hbm.at[pl.ds(0, CH)], rows[b],
                                  gsem[b]).wait()

        def scale(b):
            r = rows[b]
            for g in range(CH // L):
                w16 = w_v[b][pl.ds(g * L, L)]
                for jl in range(L):
                    j = g * L + jl
                    wj = w16.at[jnp.full((L,), jl, dtype=i32)].get(
                        mode="promise_in_bounds")
                    for cg in range(D // L):
                        sl = pl.ds(cg * L, L)
                        r[j, sl] = r[j, sl] * wj

        def scatter_start(b):
            pltpu.async_copy(rows[b], sh_acc.at[dst_v[b]], ssem[b],
                             add=True)

        def scatter_wait(b):
            pltpu.make_async_copy(y_hbm.at[pl.ds(0, CH)], rows[b],
                                  ssem[b]).wait()

        for b in range(min(2, NIT)):
            idx_start(b, b)
            idx_wait(b)
            gather_start(b)

        def step(ci, b):
            nxt = ci + 2
            bn = (b + 2) % NB

            @pl.when(nxt < NIT)
            def _():
                @pl.when(ci >= 2)
                def _():
                    scatter_wait(bn)

                idx_start(nxt, bn)
                idx_wait(bn)
                gather_start(bn)

            gather_wait(b)
            scale(b)
            scatter_start(b)

        def body(k, carry):
            for b in range(NB):
                step(k * NB + b, b)
            return carry

        lax.fori_loop(0, NIT // NB, body, 0)
        for t in range(NIT % NB):
            ci = (NIT // NB) * NB + t
            step(ci, ci % NB)
        for b in range(NB):
            scatter_wait(b)
        plsc.subcore_barrier()
        pltpu.sync_copy(sh_acc.at[pl.ds(s * RPT, RPT)],
                        out_hbm.at[c, pl.ds(s * RPT, RPT)])

    return agg_kernel


def _tc_first_body(degp_ref, x_ref, w_ref, y_ref):
    dinv = lax.rsqrt(degp_ref[0, :] + degp_ref[1, :] + 2.0)
    xw = jnp.dot(x_ref[...], w_ref[...], preferred_element_type=f32)
    y_ref[...] = xw * dinv[:, None]


def _tc_mid_body(degp_ref, a0_ref, a1_ref, y_ref, b_ref, w_ref, y1_ref):
    dinv = lax.rsqrt(degp_ref[0, :] + degp_ref[1, :] + 2.0)[:, None]
    pre = dinv * (a0_ref[...] + a1_ref[...] + 2.0 * y_ref[...]) + b_ref[...]
    h = jnp.maximum(pre, 0.0)
    xw = jnp.dot(h, w_ref[...], preferred_element_type=f32)
    y1_ref[...] = xw * dinv


def _tc_last_body(degp_ref, a0_ref, a1_ref, y_ref, b_ref, out_ref):
    dinv = lax.rsqrt(degp_ref[0, :] + degp_ref[1, :] + 2.0)[:, None]
    pre = dinv * (a0_ref[...] + a1_ref[...] + 2.0 * y_ref[...]) + b_ref[...]
    out_ref[...] = jnp.maximum(pre, 0.0)


def _row_specs(D, BR):
    deg_spec = pl.BlockSpec((NC, BR), lambda i: (0, i))
    mat_spec = pl.BlockSpec((BR, D), lambda i: (i, 0))
    w_spec = pl.BlockSpec((D, D), lambda i: (0, 0))
    b_spec = pl.BlockSpec((1, D), lambda i: (0, 0))
    return deg_spec, mat_spec, w_spec, b_spec


def _tc_first(degp, x_p, W, NP, D, BR):
    deg_spec, mat_spec, w_spec, _ = _row_specs(D, BR)
    return pl.pallas_call(
        _tc_first_body,
        grid=(NP // BR,),
        in_specs=[deg_spec, mat_spec, w_spec],
        out_specs=mat_spec,
        out_shape=jax.ShapeDtypeStruct((NP, D), f32),
    )(degp, x_p, W)


def _tc_mid(degp, a0, a1, y, b2d, W, NP, D, BR):
    deg_spec, mat_spec, w_spec, b_spec = _row_specs(D, BR)
    return pl.pallas_call(
        _tc_mid_body,
        grid=(NP // BR,),
        in_specs=[deg_spec, mat_spec, mat_spec, mat_spec, b_spec, w_spec],
        out_specs=mat_spec,
        out_shape=jax.ShapeDtypeStruct((NP, D), f32),
    )(degp, a0, a1, y, b2d, W)


def _tc_last(degp, a0, a1, y, b2d, NP, D, BR):
    deg_spec, mat_spec, _, b_spec = _row_specs(D, BR)
    return pl.pallas_call(
        _tc_last_body,
        grid=(NP // BR,),
        in_specs=[deg_spec, mat_spec, mat_spec, mat_spec, b_spec],
        out_specs=mat_spec,
        out_shape=jax.ShapeDtypeStruct((NP, D), f32),
    )(degp, a0, a1, y, b2d)


@jax.jit
def kernel(x, edge_index, edge_attr, W0, b0, W1, b1):
    N, D = x.shape
    E = edge_attr.shape[0]
    CH = 80
    NP = -(-N // (NS * 40)) * (NS * 40)
    BR = 512
    EPT = E // NW
    NIT = EPT // CH

    src = edge_index[0]
    dst = edge_index[1]
    x_p = jnp.pad(x, ((0, NP - N), (0, 0)))
    zeros1 = jnp.zeros((NP,), f32)
    zeros2 = jnp.zeros((NP // NS, D), f32)

    deg_kernel = _make_deg_kernel(E, NP, CH)
    agg_kernel = _make_agg_kernel(E, NP, D, CH)

    degp = deg_kernel(dst, edge_attr, zeros1)
    y0 = _tc_first(degp, x_p, W0, NP, D, BR)
    accp0 = agg_kernel(y0, src, dst, edge_attr, zeros2)
    y1 = _tc_mid(degp, accp0[0], accp0[1], y0,
                 b0.reshape(1, D), W1, NP, D, BR)
    accp1 = agg_kernel(y1, src, dst, edge_attr, zeros2)
    out = _tc_last(degp, accp1[0], accp1[1], y1,
                   b1.reshape(1, D), NP, D, BR)
    return out[:N]
